# baseline (speedup 1.0000x reference)
import functools

import jax
import jax.numpy as jnp
from jax.experimental import pallas as pl
from jax.experimental.pallas import tpu as pltpu

B = 128
NQ = 2048
NC = 16
NW = 20


def _dense_body(boxes_ref, seq_ref, cap_ref, cnt_ref, ts_ref,
                boxes_out, cap_out, len_out):
    c = boxes_ref[:, :, 0]
    l = boxes_ref[:, :, 1]
    ts = ts_ref[0, 0]
    x0 = jnp.clip(c - 0.5 * l, 0.0, 1.0) * ts
    x1 = jnp.clip(c + 0.5 * l, 0.0, 1.0) * ts
    boxes_out[...] = jnp.stack([x0, x1], axis=-1)
    mask = (seq_ref[...] > 0).astype(jnp.float32)
    cap_out[...] = (mask * cap_ref[...]).sum(axis=2, keepdims=True).reshape(1, 1, NQ)
    am = jnp.argmax(cnt_ref[...], axis=-1).astype(jnp.int32)
    len_out[...] = jnp.maximum(am, 1).reshape(1, 1, 1)


def _dense(pred_boxes, seq, cap_prob, pred_count, target_sizes):
    ts2 = target_sizes.reshape(B, 1, 1)
    cnt3 = pred_count.reshape(B, 1, NQ)
    grid = (B,)
    return pl.pallas_call(
        _dense_body,
        grid=grid,
        in_specs=[
            pl.BlockSpec((1, NQ, 2), lambda b: (b, 0, 0)),
            pl.BlockSpec((1, NQ, NW), lambda b: (b, 0, 0)),
            pl.BlockSpec((1, NQ, NW), lambda b: (b, 0, 0)),
            pl.BlockSpec((1, 1, NQ), lambda b: (b, 0, 0)),
            pl.BlockSpec((1, 1, 1), lambda b: (b, 0, 0)),
        ],
        out_specs=[
            pl.BlockSpec((1, NQ, 2), lambda b: (b, 0, 0)),
            pl.BlockSpec((1, 1, NQ), lambda b: (b, 0, 0)),
            pl.BlockSpec((1, 1, 1), lambda b: (b, 0, 0)),
        ],
        out_shape=[
            jax.ShapeDtypeStruct((B, NQ, 2), jnp.float32),
            jax.ShapeDtypeStruct((B, 1, NQ), jnp.float32),
            jax.ShapeDtypeStruct((B, 1, 1), jnp.int32),
        ],
    )(pred_boxes, seq, cap_prob, cnt3, ts2)


def kernel(pred_logits, pred_boxes, seq, cap_prob, pred_count, target_sizes):
    boxes_s, cap_full, lens2d = _dense(pred_boxes, seq, cap_prob,
                                       pred_count, target_sizes)
    flat = pred_logits.reshape(B, NQ * NC)
    topv, topi = jax.lax.top_k(flat, NQ)
    scores = jax.nn.sigmoid(topv)
    topk_boxes = topi // NC
    labels = topi % NC
    boxes = jnp.take_along_axis(
        boxes_s, jnp.broadcast_to(topk_boxes[..., None], (B, NQ, 2)), axis=1)
    cap_scores = jnp.take_along_axis(cap_full.reshape(B, NQ), topk_boxes, axis=1)
    eseq_lens = lens2d.reshape(B)
    return scores, labels, boxes, cap_scores, eseq_lens


# speedup vs baseline: 1.8630x; 1.8630x over previous
import functools

import jax
import jax.numpy as jnp
from jax import lax
from jax.experimental import pallas as pl
from jax.experimental.pallas import tpu as pltpu
from jax.experimental.pallas import tpu_sc as plsc

B = 128
NQ = 2048
NCLS = 16
NWORD = 20
NFLAT = NQ * NCLS
NWORKERS = 32
ROWS_PER_W = B // NWORKERS

MIN32 = -2147483648
M7F8 = 0x7F800000


def _dense_body(boxes_ref, seq_ref, cap_ref, cnt_ref, ts_ref,
                boxes_out, cap_out, len_out):
    c = boxes_ref[:, :, 0]
    l = boxes_ref[:, :, 1]
    ts = ts_ref[0, 0]
    x0 = jnp.clip(c - 0.5 * l, 0.0, 1.0) * ts
    x1 = jnp.clip(c + 0.5 * l, 0.0, 1.0) * ts
    boxes_out[...] = jnp.stack([x0, x1], axis=-1)
    mask = (seq_ref[...] > 0).astype(jnp.float32)
    cap_out[...] = (mask * cap_ref[...]).sum(axis=2, keepdims=True).reshape(1, 1, NQ)
    am = jnp.argmax(cnt_ref[...], axis=-1).astype(jnp.int32)
    len_out[...] = jnp.maximum(am, 1).reshape(1, 1, 1)


def _dense(pred_boxes, seq, cap_prob, pred_count, target_sizes):
    ts2 = target_sizes.reshape(B, 1, 1)
    cnt3 = pred_count.reshape(B, 1, NQ)
    return pl.pallas_call(
        _dense_body,
        grid=(B,),
        in_specs=[
            pl.BlockSpec((1, NQ, 2), lambda b: (b, 0, 0)),
            pl.BlockSpec((1, NQ, NWORD), lambda b: (b, 0, 0)),
            pl.BlockSpec((1, NQ, NWORD), lambda b: (b, 0, 0)),
            pl.BlockSpec((1, 1, NQ), lambda b: (b, 0, 0)),
            pl.BlockSpec((1, 1, 1), lambda b: (b, 0, 0)),
        ],
        out_specs=[
            pl.BlockSpec((1, NQ, 2), lambda b: (b, 0, 0)),
            pl.BlockSpec((1, 1, NQ), lambda b: (b, 0, 0)),
            pl.BlockSpec((1, 1, 1), lambda b: (b, 0, 0)),
        ],
        out_shape=[
            jax.ShapeDtypeStruct((B, NQ, 2), jnp.float32),
            jax.ShapeDtypeStruct((B, 1, NQ), jnp.float32),
            jax.ShapeDtypeStruct((B, 1, 1), jnp.int32),
        ],
    )(pred_boxes, seq, cap_prob, cnt3, ts2)


_IOTA = None


def _clear256(ref):
    for c in range(16):
        ref[pl.ds(c * 16, 16)] = jnp.zeros((16,), jnp.int32)


def _find_threshold(hist, k_rem, iota16):
    def chunk(t, carry):
        cum, dstar, above, done = carry
        c = 15 - t
        v = hist[pl.ds(c * 16, 16)]
        rv = lax.rev(v, (0,))
        inc = jnp.cumsum(rv)
        sfx = cum + inc
        found = sfx >= k_rem
        j = jnp.min(jnp.where(found, iota16, 16))
        hit = jnp.logical_and(done == 0, j < 16)
        d_new = c * 16 + 15 - j
        above_new = cum + jnp.sum(jnp.where(iota16 == j, inc - rv, 0))
        dstar = jnp.where(hit, d_new, dstar)
        above = jnp.where(hit, above_new, above)
        done = jnp.where(j < 16, jnp.int32(1), done)
        cum = cum + jnp.sum(v)
        return cum, dstar, above, done
    _, dstar, above, _ = lax.fori_loop(
        0, 16, chunk,
        (jnp.int32(0), jnp.int32(0), jnp.int32(0), jnp.int32(0)))
    return dstar, above


def _sc_body(bits_hbm, boxin_hbm, capin_hbm,
             scores_hbm, labels_hbm, boxes_hbm, caps_hbm,
             keys, ckey, cidx, dkey, didx, hist, hbase,
             boxrow, caprow, srow, lrow, brow, crow):
    cid = lax.axis_index("c")
    sid = lax.axis_index("s")
    wid = sid * 2 + cid
    iota16 = lax.iota(jnp.int32, 16)

    def do_row(t, carry):
        row = wid * ROWS_PER_W + t
        pltpu.sync_copy(bits_hbm.at[row], keys)
        pltpu.sync_copy(boxin_hbm.at[row], boxrow)
        pltpu.sync_copy(capin_hbm.at[row], caprow)

        _clear256(hist)

        def p0(i, cc):
            b = keys[pl.ds(i * 16, 16)]
            k = jnp.where(b >= 0, b, M7F8 - b)
            u = k ^ MIN32
            keys[pl.ds(i * 16, 16)] = u
            d = (u >> 24) & 0xFF
            cnt, last = plsc.scan_count(d)
            plsc.addupdate_scatter(hist, [d], cnt, mask=last)
            return cc
        lax.fori_loop(0, NFLAT // 16, p0, 0)

        k_rem = jnp.int32(NQ)
        dstar, above = _find_threshold(hist, k_rem, iota16)
        prefix = dstar << 24
        k_rem = k_rem - above

        for p in range(1, 4):
            shift = 24 - 8 * p
            _clear256(hist)
            pfx_hi = prefix >> (shift + 8)

            def hp(i, cc, _shift=shift, _pfx=pfx_hi):
                u = keys[pl.ds(i * 16, 16)]
                act = (u >> (_shift + 8)) == _pfx
                d = (u >> _shift) & 0xFF
                cnt, last = plsc.scan_count(d, mask=act)
                plsc.addupdate_scatter(hist, [d], cnt, mask=last)
                return cc
            lax.fori_loop(0, NFLAT // 16, hp, 0)
            dstar, above = _find_threshold(hist, k_rem, iota16)
            prefix = prefix | (dstar << shift)
            k_rem = k_rem - above

        t_u = prefix
        t_s = t_u ^ MIN32
        g_total = jnp.int32(NQ) - k_rem

        def ex(i, carry):
            bg, be = carry
            u = keys[pl.ds(i * 16, 16)]
            su = u ^ MIN32
            mg = su > t_s
            me = jnp.logical_and(u == t_u, be < NQ)
            pos = i * 16 + iota16
            plsc.store_compressed(ckey.at[pl.ds(bg, 16)], u, mask=mg)
            plsc.store_compressed(cidx.at[pl.ds(bg, 16)], pos, mask=mg)
            plsc.store_compressed(ckey.at[pl.ds(be, 16)], u, mask=me)
            plsc.store_compressed(cidx.at[pl.ds(be, 16)], pos, mask=me)
            bg = bg + jnp.sum(mg.astype(jnp.int32))
            be = be + jnp.sum(me.astype(jnp.int32))
            return bg, be
        lax.fori_loop(0, NFLAT // 16, ex, (jnp.int32(0), g_total))

        bufs = [(ckey, cidx), (dkey, didx)]
        for p in range(4):
            shift = 8 * p
            src_k, src_i = bufs[p % 2]
            dst_k, dst_i = bufs[(p + 1) % 2]
            _clear256(hist)

            def hs(i, cc, _shift=shift, _sk=src_k):
                u = _sk[pl.ds(i * 16, 16)]
                d = (u >> _shift) & 0xFF
                cnt, last = plsc.scan_count(d)
                plsc.addupdate_scatter(hist, [d], cnt, mask=last)
                return cc
            lax.fori_loop(0, NQ // 16, hs, 0)

            def off(tt, cum):
                c = 15 - tt
                v = hist[pl.ds(c * 16, 16)]
                rv = lax.rev(v, (0,))
                inc = jnp.cumsum(rv)
                base_rev = cum + inc - rv
                hbase[pl.ds(c * 16, 16)] = lax.rev(base_rev, (0,))
                return cum + jnp.sum(v)
            lax.fori_loop(0, 16, off, jnp.int32(0))

            def perm(i, cc, _shift=shift, _sk=src_k, _si=src_i,
                     _dk=dst_k, _di=dst_i):
                u = _sk[pl.ds(i * 16, 16)]
                ix = _si[pl.ds(i * 16, 16)]
                d = (u >> _shift) & 0xFF
                base = plsc.load_gather(hbase, [d])
                cnt, last = plsc.scan_count(d)
                dest = base + cnt - 1
                plsc.store_scatter(_dk, [dest], u)
                plsc.store_scatter(_di, [dest], ix)
                plsc.addupdate_scatter(hbase, [d], cnt, mask=last)
                return cc
            lax.fori_loop(0, NQ // 16, perm, 0)

        def post(i, cc):
            u = ckey[pl.ds(i * 16, 16)]
            ix = cidx[pl.ds(i * 16, 16)]
            k = u ^ MIN32
            b = jnp.where(k >= 0, k, M7F8 - k)
            x = plsc.bitcast(b, jnp.float32)
            srow[pl.ds(i * 16, 16)] = 1.0 / (1.0 + jnp.exp(-x))
            lrow[pl.ds(i * 16, 16)] = ix & (NCLS - 1)
            q = ix >> 4
            b0 = plsc.load_gather(boxrow, [2 * q])
            b1 = plsc.load_gather(boxrow, [2 * q + 1])
            pos = i * 16 + iota16
            plsc.store_scatter(brow, [2 * pos], b0)
            plsc.store_scatter(brow, [2 * pos + 1], b1)
            crow[pl.ds(i * 16, 16)] = plsc.load_gather(caprow, [q])
            return cc
        lax.fori_loop(0, NQ // 16, post, 0)

        pltpu.sync_copy(srow, scores_hbm.at[row])
        pltpu.sync_copy(lrow, labels_hbm.at[row])
        pltpu.sync_copy(brow, boxes_hbm.at[row])
        pltpu.sync_copy(crow, caps_hbm.at[row])
        return carry

    lax.fori_loop(0, ROWS_PER_W, do_row, 0)


def _topk_sc(bits, boxes_flat, cap_flat):
    mesh = plsc.VectorSubcoreMesh(core_axis_name="c", subcore_axis_name="s")
    fn = pl.kernel(
        _sc_body,
        out_type=[
            jax.ShapeDtypeStruct((B, NQ), jnp.float32),
            jax.ShapeDtypeStruct((B, NQ), jnp.int32),
            jax.ShapeDtypeStruct((B, 2 * NQ), jnp.float32),
            jax.ShapeDtypeStruct((B, NQ), jnp.float32),
        ],
        mesh=mesh,
        scratch_types=[
            pltpu.VMEM((NFLAT,), jnp.int32),
            pltpu.VMEM((NQ + 16,), jnp.int32),
            pltpu.VMEM((NQ + 16,), jnp.int32),
            pltpu.VMEM((NQ,), jnp.int32),
            pltpu.VMEM((NQ,), jnp.int32),
            pltpu.VMEM((256,), jnp.int32),
            pltpu.VMEM((256,), jnp.int32),
            pltpu.VMEM((2 * NQ,), jnp.float32),
            pltpu.VMEM((NQ,), jnp.float32),
            pltpu.VMEM((NQ,), jnp.float32),
            pltpu.VMEM((NQ,), jnp.int32),
            pltpu.VMEM((2 * NQ,), jnp.float32),
            pltpu.VMEM((NQ,), jnp.float32),
        ],
        compiler_params=pltpu.CompilerParams(needs_layout_passes=False),
    )
    return fn(bits, boxes_flat, cap_flat)


def kernel(pred_logits, pred_boxes, seq, cap_prob, pred_count, target_sizes):
    boxes_s, cap_full, lens3d = _dense(pred_boxes, seq, cap_prob,
                                       pred_count, target_sizes)
    bits = lax.bitcast_convert_type(pred_logits, jnp.int32).reshape(B, NFLAT)
    scores, labels, boxes_flat, cap_scores = _topk_sc(
        bits, boxes_s.reshape(B, 2 * NQ), cap_full.reshape(B, NQ))
    boxes = boxes_flat.reshape(B, NQ, 2)
    eseq_lens = lens3d.reshape(B)
    return scores, labels, boxes, cap_scores, eseq_lens


# speedup vs baseline: 2.7947x; 1.5001x over previous
import functools

import jax
import jax.numpy as jnp
from jax import lax
from jax.experimental import pallas as pl
from jax.experimental.pallas import tpu as pltpu
from jax.experimental.pallas import tpu_sc as plsc

B = 128
NQ = 2048
NCLS = 16
NWORD = 20
NFLAT = NQ * NCLS
NWORKERS = 32
ROWS_PER_W = B // NWORKERS
NV = NFLAT // 16
NVQ = NV // 4
SV = NQ // 16
SVQ = SV // 4

MIN32 = -2147483648
M7F8 = 0x7F800000


RB = 8
SEG = 640
NSEG = NQ * NWORD // SEG


def _dense_body(boxes_ref, seq_ref, cap_ref, cnt_ref, ts_ref,
                boxes_out, cap_out, len_out):
    m = jnp.where(seq_ref[...] > 0, cap_ref[...], 0.0)
    e = lax.broadcasted_iota(jnp.int32, (SEG, SEG // NWORD), 0)
    o = lax.broadcasted_iota(jnp.int32, (SEG, SEG // NWORD), 1)
    a = (e // NWORD == o).astype(jnp.float32)
    s = jnp.dot(m.reshape(RB * NSEG, SEG), a,
                precision=lax.Precision.HIGHEST,
                preferred_element_type=jnp.float32)
    cap_out[...] = s.reshape(RB, NSEG, SEG // NWORD)

    v = boxes_ref[...]
    nxt = jnp.roll(v, -1, axis=1)
    prv = jnp.roll(v, 1, axis=1)
    ts = ts_ref[...]
    x0 = jnp.clip(v - 0.5 * nxt, 0.0, 1.0)
    x1 = jnp.clip(prv + 0.5 * v, 0.0, 1.0)
    par = lax.broadcasted_iota(jnp.int32, (RB, 2 * NQ), 1) % 2
    boxes_out[...] = jnp.where(par == 0, x0, x1) * ts

    am = jnp.argmax(cnt_ref[...], axis=-1).astype(jnp.int32)
    len_out[...] = jnp.maximum(am, 1).reshape(RB, 1)


def _dense(pred_boxes, seq, cap_prob, pred_count, target_sizes):
    ts2 = target_sizes.reshape(B, 1)
    boxes_flat = pred_boxes.reshape(B, 2 * NQ)
    seq3 = seq.reshape(B, NSEG, SEG)
    cap3 = cap_prob.reshape(B, NSEG, SEG)
    return pl.pallas_call(
        _dense_body,
        grid=(B // RB,),
        in_specs=[
            pl.BlockSpec((RB, 2 * NQ), lambda b: (b, 0)),
            pl.BlockSpec((RB, NSEG, SEG), lambda b: (b, 0, 0)),
            pl.BlockSpec((RB, NSEG, SEG), lambda b: (b, 0, 0)),
            pl.BlockSpec((RB, NQ), lambda b: (b, 0)),
            pl.BlockSpec((RB, 1), lambda b: (b, 0)),
        ],
        out_specs=[
            pl.BlockSpec((RB, 2 * NQ), lambda b: (b, 0)),
            pl.BlockSpec((RB, NSEG, SEG // NWORD), lambda b: (b, 0, 0)),
            pl.BlockSpec((RB, 1), lambda b: (b, 0)),
        ],
        out_shape=[
            jax.ShapeDtypeStruct((B, 2 * NQ), jnp.float32),
            jax.ShapeDtypeStruct((B, NSEG, SEG // NWORD), jnp.float32),
            jax.ShapeDtypeStruct((B, 1), jnp.int32),
        ],
    )(boxes_flat, seq3, cap3, pred_count, ts2)


def _clear256(ref):
    for c in range(16):
        ref[pl.ds(c * 16, 16)] = jnp.zeros((16,), jnp.int32)


def _find_threshold(hs, k_rem, iota16):
    def chunk(t, carry):
        cum, dstar, above, done = carry
        c = 15 - t
        v = hs[0][pl.ds(c * 16, 16)]
        for q in range(1, 4):
            v = v + hs[q][pl.ds(c * 16, 16)]
        rv = lax.rev(v, (0,))
        inc = jnp.cumsum(rv)
        sfx = cum + inc
        found = sfx >= k_rem
        j = jnp.min(jnp.where(found, iota16, 16))
        hit = jnp.logical_and(done == 0, j < 16)
        d_new = c * 16 + 15 - j
        above_new = cum + jnp.sum(jnp.where(iota16 == j, inc - rv, 0))
        dstar = jnp.where(hit, d_new, dstar)
        above = jnp.where(hit, above_new, above)
        done = jnp.where(j < 16, jnp.int32(1), done)
        cum = cum + jnp.sum(v)
        return cum, dstar, above, done
    _, dstar, above, _ = lax.fori_loop(
        0, 16, chunk,
        (jnp.int32(0), jnp.int32(0), jnp.int32(0), jnp.int32(0)))
    return dstar, above


def _sc_body(bits_hbm, boxin_hbm, capin_hbm,
             scores_hbm, labels_hbm, boxes_hbm, caps_hbm,
             keys, ckey, cidx, dkey, didx,
             h0, h1, h2, h3, b0, b1, b2, b3,
             boxrow, caprow, srow, lrow, brow, crow):
    cid = lax.axis_index("c")
    sid = lax.axis_index("s")
    wid = sid * 2 + cid
    iota16 = lax.iota(jnp.int32, 16)
    hs = (h0, h1, h2, h3)
    bs = (b0, b1, b2, b3)

    def do_row(t, carry):
        row = wid * ROWS_PER_W + t
        pltpu.sync_copy(bits_hbm.at[row], keys)
        pltpu.sync_copy(boxin_hbm.at[row], boxrow)
        pltpu.sync_copy(capin_hbm.at[row], caprow)

        for q in range(4):
            _clear256(hs[q])

        def p0(i, cc):
            for q in range(4):
                off = (q * NVQ + i) * 16
                b = keys[pl.ds(off, 16)]
                k = jnp.where(b >= 0, b, M7F8 - b)
                u = k ^ MIN32
                keys[pl.ds(off, 16)] = u
                d = (u >> 24) & 0xFF
                cnt, last = plsc.scan_count(d)
                plsc.addupdate_scatter(hs[q], [d], cnt, mask=last)
            return cc
        lax.fori_loop(0, NVQ, p0, 0)

        k_rem = jnp.int32(NQ)
        dstar, above = _find_threshold(hs, k_rem, iota16)
        prefix = dstar << 24
        k_rem = k_rem - above

        for p in range(1, 4):
            shift = 24 - 8 * p
            for q in range(4):
                _clear256(hs[q])
            pfx_hi = prefix >> (shift + 8)

            def hp(i, cc, _shift=shift, _pfx=pfx_hi):
                for q in range(4):
                    off = (q * NVQ + i) * 16
                    u = keys[pl.ds(off, 16)]
                    act = (u >> (_shift + 8)) == _pfx
                    d = (u >> _shift) & 0xFF
                    cnt, last = plsc.scan_count(d, mask=act)
                    plsc.addupdate_scatter(hs[q], [d], cnt, mask=last)
                return cc
            lax.fori_loop(0, NVQ, hp, 0)
            dstar, above = _find_threshold(hs, k_rem, iota16)
            prefix = prefix | (dstar << shift)
            k_rem = k_rem - above

        t_u = prefix
        t_s = t_u ^ MIN32
        g_total = jnp.int32(NQ) - k_rem

        zeros16 = jnp.zeros((16,), jnp.int32)

        def ex(i, carry):
            bg, be = carry
            for j in range(2):
                off = (i * 2 + j) * 16
                u = keys[pl.ds(off, 16)]
                su = u ^ MIN32
                mg = su > t_s
                me = jnp.logical_and(u == t_u, be < NQ)
                pos = off + iota16
                destg = bg + jnp.cumsum(mg.astype(jnp.int32)) - 1
                plsc.store_scatter(ckey, [destg], u, mask=mg)
                plsc.store_scatter(cidx, [destg], pos, mask=mg)
                deste = be + jnp.cumsum(me.astype(jnp.int32)) - 1
                plsc.store_scatter(ckey, [deste], u, mask=me)
                plsc.store_scatter(cidx, [deste], pos, mask=me)
                bg = bg + plsc.all_reduce_population_count(mg)
                be = be + plsc.all_reduce_population_count(me)
            return bg, be
        lax.fori_loop(0, NV // 2, ex, (zeros16, zeros16 + g_total))

        bufs = [(ckey, cidx), (dkey, didx)]
        for p in range(4):
            shift = 8 * p
            src_k, src_i = bufs[p % 2]
            dst_k, dst_i = bufs[(p + 1) % 2]
            for q in range(4):
                _clear256(hs[q])

            def hsz(i, cc, _shift=shift, _sk=src_k):
                for q in range(4):
                    off = (q * SVQ + i) * 16
                    u = _sk[pl.ds(off, 16)]
                    d = (u >> _shift) & 0xFF
                    cnt, last = plsc.scan_count(d)
                    plsc.addupdate_scatter(hs[q], [d], cnt, mask=last)
                return cc
            lax.fori_loop(0, SVQ, hsz, 0)

            def offl(tt, cum):
                c = 15 - tt
                v0 = h0[pl.ds(c * 16, 16)]
                v1 = h1[pl.ds(c * 16, 16)]
                v2 = h2[pl.ds(c * 16, 16)]
                v3 = h3[pl.ds(c * 16, 16)]
                tot = v0 + v1 + v2 + v3
                rtot = lax.rev(tot, (0,))
                inc = jnp.cumsum(rtot)
                base = lax.rev(cum + inc - rtot, (0,))
                b0[pl.ds(c * 16, 16)] = base
                b1[pl.ds(c * 16, 16)] = base + v0
                b2[pl.ds(c * 16, 16)] = base + v0 + v1
                b3[pl.ds(c * 16, 16)] = base + v0 + v1 + v2
                return cum + jnp.sum(tot)
            lax.fori_loop(0, 16, offl, jnp.int32(0))

            def perm(i, cc, _shift=shift, _sk=src_k, _si=src_i,
                     _dk=dst_k, _di=dst_i):
                for q in range(4):
                    off = (q * SVQ + i) * 16
                    u = _sk[pl.ds(off, 16)]
                    ix = _si[pl.ds(off, 16)]
                    d = (u >> _shift) & 0xFF
                    base = plsc.load_gather(bs[q], [d])
                    cnt, last = plsc.scan_count(d)
                    dest = base + cnt - 1
                    plsc.store_scatter(_dk, [dest], u)
                    plsc.store_scatter(_di, [dest], ix)
                    plsc.addupdate_scatter(bs[q], [d], cnt, mask=last)
                return cc
            lax.fori_loop(0, SVQ, perm, 0)

        def post(i, cc):
            for j in range(2):
                off = (i * 2 + j) * 16
                u = ckey[pl.ds(off, 16)]
                ix = cidx[pl.ds(off, 16)]
                k = u ^ MIN32
                b = jnp.where(k >= 0, k, M7F8 - k)
                x = plsc.bitcast(b, jnp.float32)
                srow[pl.ds(off, 16)] = 1.0 / (1.0 + jnp.exp(-x))
                lrow[pl.ds(off, 16)] = ix & (NCLS - 1)
                q2 = ix >> 4
                bx0 = plsc.load_gather(boxrow, [2 * q2])
                bx1 = plsc.load_gather(boxrow, [2 * q2 + 1])
                pos = off + iota16
                plsc.store_scatter(brow, [2 * pos], bx0)
                plsc.store_scatter(brow, [2 * pos + 1], bx1)
                crow[pl.ds(off, 16)] = plsc.load_gather(caprow, [q2])
            return cc
        lax.fori_loop(0, SV // 2, post, 0)

        pltpu.sync_copy(srow, scores_hbm.at[row])
        pltpu.sync_copy(lrow, labels_hbm.at[row])
        pltpu.sync_copy(brow, boxes_hbm.at[row])
        pltpu.sync_copy(crow, caps_hbm.at[row])
        return carry

    lax.fori_loop(0, ROWS_PER_W, do_row, 0)


def _topk_sc(bits, boxes_flat, cap_flat):
    mesh = plsc.VectorSubcoreMesh(core_axis_name="c", subcore_axis_name="s")
    fn = pl.kernel(
        _sc_body,
        out_type=[
            jax.ShapeDtypeStruct((B, NQ), jnp.float32),
            jax.ShapeDtypeStruct((B, NQ), jnp.int32),
            jax.ShapeDtypeStruct((B, 2 * NQ), jnp.float32),
            jax.ShapeDtypeStruct((B, NQ), jnp.float32),
        ],
        mesh=mesh,
        scratch_types=[
            pltpu.VMEM((NFLAT,), jnp.int32),
            pltpu.VMEM((NQ + 16,), jnp.int32),
            pltpu.VMEM((NQ + 16,), jnp.int32),
            pltpu.VMEM((NQ,), jnp.int32),
            pltpu.VMEM((NQ,), jnp.int32),
        ] + [pltpu.VMEM((256,), jnp.int32)] * 8 + [
            pltpu.VMEM((2 * NQ,), jnp.float32),
            pltpu.VMEM((NQ,), jnp.float32),
            pltpu.VMEM((NQ,), jnp.float32),
            pltpu.VMEM((NQ,), jnp.int32),
            pltpu.VMEM((2 * NQ,), jnp.float32),
            pltpu.VMEM((NQ,), jnp.float32),
        ],
        compiler_params=pltpu.CompilerParams(needs_layout_passes=False),
    )
    return fn(bits, boxes_flat, cap_flat)


def kernel(pred_logits, pred_boxes, seq, cap_prob, pred_count, target_sizes):
    boxes_s, cap_full, lens2d = _dense(pred_boxes, seq, cap_prob,
                                       pred_count, target_sizes)
    bits = lax.bitcast_convert_type(pred_logits, jnp.int32).reshape(B, NFLAT)
    scores, labels, boxes_flat, cap_scores = _topk_sc(
        bits, boxes_s, cap_full.reshape(B, NQ))
    boxes = boxes_flat.reshape(B, NQ, 2)
    eseq_lens = lens2d.reshape(B)
    return scores, labels, boxes, cap_scores, eseq_lens


# speedup vs baseline: 3.5633x; 1.2750x over previous
import functools

import jax
import jax.numpy as jnp
from jax import lax
from jax.experimental import pallas as pl
from jax.experimental.pallas import tpu as pltpu
from jax.experimental.pallas import tpu_sc as plsc

B = 128
NQ = 2048
NCLS = 16
NWORD = 20
NFLAT = NQ * NCLS
NWORKERS = 32
ROWS_PER_W = B // NWORKERS
NV = NFLAT // 16
NVQ = NV // 4
SV = NQ // 16
SVQ = SV // 4

MIN32 = -2147483648
M7F8 = 0x7F800000


RB = 8
SEG = 640
NSEG = NQ * NWORD // SEG


def _dense_body(boxes_ref, seq_ref, cap_ref, cnt_ref, ts_ref,
                boxes_out, cap_out, len_out):
    m = jnp.where(seq_ref[...] > 0, cap_ref[...], 0.0)
    e = lax.broadcasted_iota(jnp.int32, (SEG, SEG // NWORD), 0)
    o = lax.broadcasted_iota(jnp.int32, (SEG, SEG // NWORD), 1)
    a = (e // NWORD == o).astype(jnp.float32)
    s = jnp.dot(m.reshape(RB * NSEG, SEG), a,
                precision=lax.Precision.HIGHEST,
                preferred_element_type=jnp.float32)
    cap_out[...] = s.reshape(RB, NSEG, SEG // NWORD)

    v = boxes_ref[...]
    nxt = jnp.roll(v, -1, axis=1)
    prv = jnp.roll(v, 1, axis=1)
    ts = ts_ref[...]
    x0 = jnp.clip(v - 0.5 * nxt, 0.0, 1.0)
    x1 = jnp.clip(prv + 0.5 * v, 0.0, 1.0)
    par = lax.broadcasted_iota(jnp.int32, (RB, 2 * NQ), 1) % 2
    boxes_out[...] = jnp.where(par == 0, x0, x1) * ts

    am = jnp.argmax(cnt_ref[...], axis=-1).astype(jnp.int32)
    len_out[...] = jnp.maximum(am, 1).reshape(RB, 1)


def _dense(pred_boxes, seq, cap_prob, pred_count, target_sizes):
    ts2 = target_sizes.reshape(B, 1)
    boxes_flat = pred_boxes.reshape(B, 2 * NQ)
    seq3 = seq.reshape(B, NSEG, SEG)
    cap3 = cap_prob.reshape(B, NSEG, SEG)
    return pl.pallas_call(
        _dense_body,
        grid=(B // RB,),
        in_specs=[
            pl.BlockSpec((RB, 2 * NQ), lambda b: (b, 0)),
            pl.BlockSpec((RB, NSEG, SEG), lambda b: (b, 0, 0)),
            pl.BlockSpec((RB, NSEG, SEG), lambda b: (b, 0, 0)),
            pl.BlockSpec((RB, NQ), lambda b: (b, 0)),
            pl.BlockSpec((RB, 1), lambda b: (b, 0)),
        ],
        out_specs=[
            pl.BlockSpec((RB, 2 * NQ), lambda b: (b, 0)),
            pl.BlockSpec((RB, NSEG, SEG // NWORD), lambda b: (b, 0, 0)),
            pl.BlockSpec((RB, 1), lambda b: (b, 0)),
        ],
        out_shape=[
            jax.ShapeDtypeStruct((B, 2 * NQ), jnp.float32),
            jax.ShapeDtypeStruct((B, NSEG, SEG // NWORD), jnp.float32),
            jax.ShapeDtypeStruct((B, 1), jnp.int32),
        ],
    )(boxes_flat, seq3, cap3, pred_count, ts2)


def _clear256(ref):
    for c in range(16):
        ref[pl.ds(c * 16, 16)] = jnp.zeros((16,), jnp.int32)


def _find_threshold(hs, k_rem, iota16):
    def chunk(t, carry):
        cum, dstar, above, done = carry
        c = 15 - t
        v = hs[0][pl.ds(c * 16, 16)]
        for q in range(1, len(hs)):
            v = v + hs[q][pl.ds(c * 16, 16)]
        rv = lax.rev(v, (0,))
        inc = jnp.cumsum(rv)
        sfx = cum + inc
        found = sfx >= k_rem
        j = jnp.min(jnp.where(found, iota16, 16))
        hit = jnp.logical_and(done == 0, j < 16)
        d_new = c * 16 + 15 - j
        above_new = cum + jnp.sum(jnp.where(iota16 == j, inc - rv, 0))
        dstar = jnp.where(hit, d_new, dstar)
        above = jnp.where(hit, above_new, above)
        done = jnp.where(j < 16, jnp.int32(1), done)
        cum = cum + jnp.sum(v)
        return cum, dstar, above, done
    _, dstar, above, _ = lax.fori_loop(
        0, 16, chunk,
        (jnp.int32(0), jnp.int32(0), jnp.int32(0), jnp.int32(0)))
    return dstar, above


def _sc_body(bits_hbm, boxin_hbm, capin_hbm,
             scores_hbm, labels_hbm, boxes_hbm, caps_hbm,
             keys, ckey, cidx, dkey, didx,
             h0, h1, h2, h3, b0, b1, b2, b3, ha, hb,
             boxrow, caprow, srow, lrow, brow, crow):
    cid = lax.axis_index("c")
    sid = lax.axis_index("s")
    wid = sid * 2 + cid
    iota16 = lax.iota(jnp.int32, 16)
    hs = (h0, h1, h2, h3)
    bs = (b0, b1, b2, b3)

    def do_row(t, carry):
        row = wid * ROWS_PER_W + t
        pltpu.sync_copy(bits_hbm.at[row], keys)
        pltpu.sync_copy(boxin_hbm.at[row], boxrow)
        pltpu.sync_copy(capin_hbm.at[row], caprow)

        ones16 = jnp.full((16,), 1, jnp.int32)

        def _clear_lane_hists():
            def clr(i, cc):
                ha[pl.ds(i * 16, 16)] = jnp.zeros((16,), jnp.int32)
                hb[pl.ds(i * 16, 16)] = jnp.zeros((16,), jnp.int32)
                return cc
            lax.fori_loop(0, 256, clr, 0)

        def _reduce_lane_hists():
            def red(c, cc):
                dbase = (c * 16 + iota16) << 4
                tot = jnp.zeros((16,), jnp.int32)
                for lane in range(16):
                    addr = dbase | (lane ^ iota16)
                    tot = tot + plsc.load_gather(ha, [addr])
                    tot = tot + plsc.load_gather(hb, [addr])
                h0[pl.ds(c * 16, 16)] = tot
                return cc
            lax.fori_loop(0, 16, red, 0)

        _clear_lane_hists()

        def p0(i, cc):
            for j in range(4):
                off = (i * 4 + j) * 16
                b = keys[pl.ds(off, 16)]
                k = jnp.where(b >= 0, b, M7F8 - b)
                u = k ^ MIN32
                keys[pl.ds(off, 16)] = u
                d = (u >> 24) & 0xFF
                addr = (d << 4) | (iota16 ^ (d & 15))
                plsc.addupdate_scatter(ha if j % 2 == 0 else hb,
                                       [addr], ones16)
            return cc
        lax.fori_loop(0, NV // 4, p0, 0)
        _reduce_lane_hists()

        k_rem = jnp.int32(NQ)
        dstar, above = _find_threshold((h0,), k_rem, iota16)
        prefix = dstar << 24
        k_rem = k_rem - above

        for p in range(1, 4):
            shift = 24 - 8 * p
            _clear_lane_hists()
            pfx_hi = prefix >> (shift + 8)

            def hp(i, cc, _shift=shift, _pfx=pfx_hi):
                for j in range(4):
                    off = (i * 4 + j) * 16
                    u = keys[pl.ds(off, 16)]
                    act = (u >> (_shift + 8)) == _pfx
                    d = (u >> _shift) & 0xFF
                    addr = (d << 4) | (iota16 ^ (d & 15))
                    plsc.addupdate_scatter(ha if j % 2 == 0 else hb,
                                           [addr], ones16, mask=act)
                return cc
            lax.fori_loop(0, NV // 4, hp, 0)
            _reduce_lane_hists()
            dstar, above = _find_threshold((h0,), k_rem, iota16)
            prefix = prefix | (dstar << shift)
            k_rem = k_rem - above

        t_u = prefix
        t_s = t_u ^ MIN32
        g_total = jnp.int32(NQ) - k_rem

        zeros16 = jnp.zeros((16,), jnp.int32)

        def ex(i, carry):
            bg, be = carry
            for j in range(2):
                off = (i * 2 + j) * 16
                u = keys[pl.ds(off, 16)]
                su = u ^ MIN32
                mg = su > t_s
                me = jnp.logical_and(u == t_u, be < NQ)
                pos = off + iota16
                packed = mg.astype(jnp.int32) + (me.astype(jnp.int32) << 16)
                cs = jnp.cumsum(packed)
                destg = bg + (cs & 0xFFFF) - 1
                deste = be + (cs >> 16) - 1
                plsc.store_scatter(ckey, [destg], u, mask=mg)
                plsc.store_scatter(cidx, [destg], pos, mask=mg)
                plsc.store_scatter(ckey, [deste], u, mask=me)
                plsc.store_scatter(cidx, [deste], pos, mask=me)
                bg = bg + plsc.all_reduce_population_count(mg)
                be = be + plsc.all_reduce_population_count(me)
            return bg, be
        lax.fori_loop(0, NV // 2, ex, (zeros16, zeros16 + g_total))

        bufs = [(ckey, cidx), (dkey, didx)]
        for p in range(4):
            shift = 8 * p
            src_k, src_i = bufs[p % 2]
            dst_k, dst_i = bufs[(p + 1) % 2]
            for q in range(4):
                _clear256(hs[q])

            def hsz(i, cc, _shift=shift, _sk=src_k):
                for q in range(4):
                    off = (q * SVQ + i) * 16
                    u = _sk[pl.ds(off, 16)]
                    d = (u >> _shift) & 0xFF
                    cnt, last = plsc.scan_count(d)
                    plsc.addupdate_scatter(hs[q], [d], cnt, mask=last)
                return cc
            lax.fori_loop(0, SVQ, hsz, 0)

            def offl(tt, cum):
                c = 15 - tt
                v0 = h0[pl.ds(c * 16, 16)]
                v1 = h1[pl.ds(c * 16, 16)]
                v2 = h2[pl.ds(c * 16, 16)]
                v3 = h3[pl.ds(c * 16, 16)]
                tot = v0 + v1 + v2 + v3
                rtot = lax.rev(tot, (0,))
                inc = jnp.cumsum(rtot)
                base = lax.rev(cum + inc - rtot, (0,))
                b0[pl.ds(c * 16, 16)] = base
                b1[pl.ds(c * 16, 16)] = base + v0
                b2[pl.ds(c * 16, 16)] = base + v0 + v1
                b3[pl.ds(c * 16, 16)] = base + v0 + v1 + v2
                return cum + jnp.sum(tot)
            lax.fori_loop(0, 16, offl, jnp.int32(0))

            def perm(i, cc, _shift=shift, _sk=src_k, _si=src_i,
                     _dk=dst_k, _di=dst_i):
                for q in range(4):
                    off = (q * SVQ + i) * 16
                    u = _sk[pl.ds(off, 16)]
                    ix = _si[pl.ds(off, 16)]
                    d = (u >> _shift) & 0xFF
                    base = plsc.load_gather(bs[q], [d])
                    cnt, last = plsc.scan_count(d)
                    dest = base + cnt - 1
                    plsc.store_scatter(_dk, [dest], u)
                    plsc.store_scatter(_di, [dest], ix)
                    plsc.addupdate_scatter(bs[q], [d], cnt, mask=last)
                return cc
            lax.fori_loop(0, SVQ, perm, 0)

        def post(i, cc):
            for j in range(2):
                off = (i * 2 + j) * 16
                u = ckey[pl.ds(off, 16)]
                ix = cidx[pl.ds(off, 16)]
                k = u ^ MIN32
                b = jnp.where(k >= 0, k, M7F8 - k)
                x = plsc.bitcast(b, jnp.float32)
                srow[pl.ds(off, 16)] = 1.0 / (1.0 + jnp.exp(-x))
                lrow[pl.ds(off, 16)] = ix & (NCLS - 1)
                q2 = ix >> 4
                bx0 = plsc.load_gather(boxrow, [2 * q2])
                bx1 = plsc.load_gather(boxrow, [2 * q2 + 1])
                pos = off + iota16
                plsc.store_scatter(brow, [2 * pos], bx0)
                plsc.store_scatter(brow, [2 * pos + 1], bx1)
                crow[pl.ds(off, 16)] = plsc.load_gather(caprow, [q2])
            return cc
        lax.fori_loop(0, SV // 2, post, 0)

        pltpu.sync_copy(srow, scores_hbm.at[row])
        pltpu.sync_copy(lrow, labels_hbm.at[row])
        pltpu.sync_copy(brow, boxes_hbm.at[row])
        pltpu.sync_copy(crow, caps_hbm.at[row])
        return carry

    lax.fori_loop(0, ROWS_PER_W, do_row, 0)


def _topk_sc(bits, boxes_flat, cap_flat):
    mesh = plsc.VectorSubcoreMesh(core_axis_name="c", subcore_axis_name="s")
    fn = pl.kernel(
        _sc_body,
        out_type=[
            jax.ShapeDtypeStruct((B, NQ), jnp.float32),
            jax.ShapeDtypeStruct((B, NQ), jnp.int32),
            jax.ShapeDtypeStruct((B, 2 * NQ), jnp.float32),
            jax.ShapeDtypeStruct((B, NQ), jnp.float32),
        ],
        mesh=mesh,
        scratch_types=[
            pltpu.VMEM((NFLAT,), jnp.int32),
            pltpu.VMEM((NQ + 16,), jnp.int32),
            pltpu.VMEM((NQ + 16,), jnp.int32),
            pltpu.VMEM((NQ,), jnp.int32),
            pltpu.VMEM((NQ,), jnp.int32),
        ] + [pltpu.VMEM((256,), jnp.int32)] * 8 + [
            pltpu.VMEM((4096,), jnp.int32),
            pltpu.VMEM((4096,), jnp.int32),
        ] + [
            pltpu.VMEM((2 * NQ,), jnp.float32),
            pltpu.VMEM((NQ,), jnp.float32),
            pltpu.VMEM((NQ,), jnp.float32),
            pltpu.VMEM((NQ,), jnp.int32),
            pltpu.VMEM((2 * NQ,), jnp.float32),
            pltpu.VMEM((NQ,), jnp.float32),
        ],
        compiler_params=pltpu.CompilerParams(needs_layout_passes=False),
    )
    return fn(bits, boxes_flat, cap_flat)


def kernel(pred_logits, pred_boxes, seq, cap_prob, pred_count, target_sizes):
    boxes_s, cap_full, lens2d = _dense(pred_boxes, seq, cap_prob,
                                       pred_count, target_sizes)
    bits = lax.bitcast_convert_type(pred_logits, jnp.int32).reshape(B, NFLAT)
    scores, labels, boxes_flat, cap_scores = _topk_sc(
        bits, boxes_s, cap_full.reshape(B, NQ))
    boxes = boxes_flat.reshape(B, NQ, 2)
    eseq_lens = lens2d.reshape(B)
    return scores, labels, boxes, cap_scores, eseq_lens


# speedup vs baseline: 3.5868x; 1.0066x over previous
import functools

import jax
import jax.numpy as jnp
from jax import lax
from jax.experimental import pallas as pl
from jax.experimental.pallas import tpu as pltpu
from jax.experimental.pallas import tpu_sc as plsc

B = 128
NQ = 2048
NCLS = 16
NWORD = 20
NFLAT = NQ * NCLS
NWORKERS = 32
ROWS_PER_W = B // NWORKERS
NV = NFLAT // 16
NVQ = NV // 4
SV = NQ // 16
SVQ = SV // 4

MIN32 = -2147483648
M7F8 = 0x7F800000


RB = 8
SEG = 640
NSEG = NQ * NWORD // SEG


def _dense_body(boxes_ref, seq_ref, cap_ref, cnt_ref, ts_ref,
                boxes_out, cap_out, len_out):
    m = jnp.where(seq_ref[...] > 0, cap_ref[...], 0.0)
    e = lax.broadcasted_iota(jnp.int32, (SEG, SEG // NWORD), 0)
    o = lax.broadcasted_iota(jnp.int32, (SEG, SEG // NWORD), 1)
    a = (e // NWORD == o).astype(jnp.float32)
    s = jnp.dot(m.reshape(RB * NSEG, SEG), a,
                precision=lax.Precision.HIGHEST,
                preferred_element_type=jnp.float32)
    cap_out[...] = s.reshape(RB, NSEG, SEG // NWORD)

    v = boxes_ref[...]
    nxt = jnp.roll(v, -1, axis=1)
    prv = jnp.roll(v, 1, axis=1)
    ts = ts_ref[...]
    x0 = jnp.clip(v - 0.5 * nxt, 0.0, 1.0)
    x1 = jnp.clip(prv + 0.5 * v, 0.0, 1.0)
    par = lax.broadcasted_iota(jnp.int32, (RB, 2 * NQ), 1) % 2
    boxes_out[...] = jnp.where(par == 0, x0, x1) * ts

    am = jnp.argmax(cnt_ref[...], axis=-1).astype(jnp.int32)
    len_out[...] = jnp.maximum(am, 1).reshape(RB, 1)


def _dense(pred_boxes, seq, cap_prob, pred_count, target_sizes):
    ts2 = target_sizes.reshape(B, 1)
    boxes_flat = pred_boxes.reshape(B, 2 * NQ)
    seq3 = seq.reshape(B, NSEG, SEG)
    cap3 = cap_prob.reshape(B, NSEG, SEG)
    return pl.pallas_call(
        _dense_body,
        grid=(B // RB,),
        in_specs=[
            pl.BlockSpec((RB, 2 * NQ), lambda b: (b, 0)),
            pl.BlockSpec((RB, NSEG, SEG), lambda b: (b, 0, 0)),
            pl.BlockSpec((RB, NSEG, SEG), lambda b: (b, 0, 0)),
            pl.BlockSpec((RB, NQ), lambda b: (b, 0)),
            pl.BlockSpec((RB, 1), lambda b: (b, 0)),
        ],
        out_specs=[
            pl.BlockSpec((RB, 2 * NQ), lambda b: (b, 0)),
            pl.BlockSpec((RB, NSEG, SEG // NWORD), lambda b: (b, 0, 0)),
            pl.BlockSpec((RB, 1), lambda b: (b, 0)),
        ],
        out_shape=[
            jax.ShapeDtypeStruct((B, 2 * NQ), jnp.float32),
            jax.ShapeDtypeStruct((B, NSEG, SEG // NWORD), jnp.float32),
            jax.ShapeDtypeStruct((B, 1), jnp.int32),
        ],
    )(boxes_flat, seq3, cap3, pred_count, ts2)


def _clear256(ref):
    for c in range(16):
        ref[pl.ds(c * 16, 16)] = jnp.zeros((16,), jnp.int32)


def _find_threshold(hs, k_rem, iota16):
    def chunk(t, carry):
        cum, dstar, above, done = carry
        c = 15 - t
        v = hs[0][pl.ds(c * 16, 16)]
        for q in range(1, len(hs)):
            v = v + hs[q][pl.ds(c * 16, 16)]
        rv = lax.rev(v, (0,))
        inc = jnp.cumsum(rv)
        sfx = cum + inc
        found = sfx >= k_rem
        j = jnp.min(jnp.where(found, iota16, 16))
        hit = jnp.logical_and(done == 0, j < 16)
        d_new = c * 16 + 15 - j
        above_new = cum + jnp.sum(jnp.where(iota16 == j, inc - rv, 0))
        dstar = jnp.where(hit, d_new, dstar)
        above = jnp.where(hit, above_new, above)
        done = jnp.where(j < 16, jnp.int32(1), done)
        cum = cum + jnp.sum(v)
        return cum, dstar, above, done
    _, dstar, above, _ = lax.fori_loop(
        0, 16, chunk,
        (jnp.int32(0), jnp.int32(0), jnp.int32(0), jnp.int32(0)))
    return dstar, above


def _sc_body(bits_hbm, boxin_hbm, capin_hbm,
             scores_hbm, labels_hbm, boxes_hbm, caps_hbm,
             keys, ckey, cidx, dkey, didx,
             h0, h1, h2, h3, b0, b1, b2, b3, ha, hb,
             boxrow, caprow, srow, lrow, brow, crow):
    cid = lax.axis_index("c")
    sid = lax.axis_index("s")
    wid = sid * 2 + cid
    iota16 = lax.iota(jnp.int32, 16)
    hs = (h0, h1, h2, h3)
    bs = (b0, b1, b2, b3)

    def do_row(t, carry):
        row = wid * ROWS_PER_W + t
        pltpu.sync_copy(bits_hbm.at[row], keys)
        pltpu.sync_copy(boxin_hbm.at[row], boxrow)
        pltpu.sync_copy(capin_hbm.at[row], caprow)

        ones16 = jnp.full((16,), 1, jnp.int32)

        def _clear_lane_hists():
            def clr(i, cc):
                ha[pl.ds(i * 16, 16)] = jnp.zeros((16,), jnp.int32)
                hb[pl.ds(i * 16, 16)] = jnp.zeros((16,), jnp.int32)
                return cc
            lax.fori_loop(0, 256, clr, 0)

        def _reduce_lane_hists():
            def red(c, cc):
                dbase = (c * 16 + iota16) << 4
                tot = jnp.zeros((16,), jnp.int32)
                for lane in range(16):
                    addr = dbase | (lane ^ iota16)
                    tot = tot + plsc.load_gather(ha, [addr])
                    tot = tot + plsc.load_gather(hb, [addr])
                h0[pl.ds(c * 16, 16)] = tot
                return cc
            lax.fori_loop(0, 16, red, 0)

        _clear_lane_hists()

        def p0(i, cc):
            for j in range(4):
                off = (i * 4 + j) * 16
                b = keys[pl.ds(off, 16)]
                k = jnp.where(b >= 0, b, M7F8 - b)
                u = k ^ MIN32
                keys[pl.ds(off, 16)] = u
                d = (u >> 24) & 0xFF
                rot = (iota16 + i * 4 + j) & 15
                addr = (d << 4) | (rot ^ (d & 15))
                plsc.addupdate_scatter(ha if j % 2 == 0 else hb,
                                       [addr], ones16)
            return cc
        lax.fori_loop(0, NV // 4, p0, 0)
        _reduce_lane_hists()

        k_rem = jnp.int32(NQ)
        dstar, above = _find_threshold((h0,), k_rem, iota16)
        prefix = dstar << 24
        k_rem = k_rem - above

        for p in range(1, 4):
            shift = 24 - 8 * p
            _clear_lane_hists()
            pfx_hi = prefix >> (shift + 8)

            def hp(i, cc, _shift=shift, _pfx=pfx_hi):
                for j in range(4):
                    off = (i * 4 + j) * 16
                    u = keys[pl.ds(off, 16)]
                    act = (u >> (_shift + 8)) == _pfx
                    d = (u >> _shift) & 0xFF
                    rot = (iota16 + i * 4 + j) & 15
                    addr = (d << 4) | (rot ^ (d & 15))
                    plsc.addupdate_scatter(ha if j % 2 == 0 else hb,
                                           [addr], ones16, mask=act)
                return cc
            lax.fori_loop(0, NV // 4, hp, 0)
            _reduce_lane_hists()
            dstar, above = _find_threshold((h0,), k_rem, iota16)
            prefix = prefix | (dstar << shift)
            k_rem = k_rem - above

        t_u = prefix
        t_s = t_u ^ MIN32
        g_total = jnp.int32(NQ) - k_rem

        zeros16 = jnp.zeros((16,), jnp.int32)

        def ex(i, carry):
            bg, be = carry
            for j in range(2):
                off = (i * 2 + j) * 16
                u = keys[pl.ds(off, 16)]
                su = u ^ MIN32
                mg = su > t_s
                me = jnp.logical_and(u == t_u, be < NQ)
                pos = off + iota16
                packed = mg.astype(jnp.int32) + (me.astype(jnp.int32) << 16)
                cs = jnp.cumsum(packed)
                destg = bg + (cs & 0xFFFF) - 1
                deste = be + (cs >> 16) - 1
                plsc.store_scatter(ckey, [destg], u, mask=mg)
                plsc.store_scatter(cidx, [destg], pos, mask=mg)
                plsc.store_scatter(ckey, [deste], u, mask=me)
                plsc.store_scatter(cidx, [deste], pos, mask=me)
                bg = bg + plsc.all_reduce_population_count(mg)
                be = be + plsc.all_reduce_population_count(me)
            return bg, be
        lax.fori_loop(0, NV // 2, ex, (zeros16, zeros16 + g_total))

        bufs = [(ckey, cidx), (dkey, didx)]
        for p in range(4):
            shift = 8 * p
            src_k, src_i = bufs[p % 2]
            dst_k, dst_i = bufs[(p + 1) % 2]
            for q in range(4):
                _clear256(hs[q])

            def hsz(i, cc, _shift=shift, _sk=src_k):
                for q in range(4):
                    off = (q * SVQ + i) * 16
                    u = _sk[pl.ds(off, 16)]
                    d = (u >> _shift) & 0xFF
                    cnt, last = plsc.scan_count(d)
                    plsc.addupdate_scatter(hs[q], [d], cnt, mask=last)
                return cc
            lax.fori_loop(0, SVQ, hsz, 0)

            def offl(tt, cum):
                c = 15 - tt
                v0 = h0[pl.ds(c * 16, 16)]
                v1 = h1[pl.ds(c * 16, 16)]
                v2 = h2[pl.ds(c * 16, 16)]
                v3 = h3[pl.ds(c * 16, 16)]
                tot = v0 + v1 + v2 + v3
                rtot = lax.rev(tot, (0,))
                inc = jnp.cumsum(rtot)
                base = lax.rev(cum + inc - rtot, (0,))
                b0[pl.ds(c * 16, 16)] = base
                b1[pl.ds(c * 16, 16)] = base + v0
                b2[pl.ds(c * 16, 16)] = base + v0 + v1
                b3[pl.ds(c * 16, 16)] = base + v0 + v1 + v2
                return cum + jnp.sum(tot)
            lax.fori_loop(0, 16, offl, jnp.int32(0))

            def perm(i, cc, _shift=shift, _sk=src_k, _si=src_i,
                     _dk=dst_k, _di=dst_i):
                for q in range(4):
                    off = (q * SVQ + i) * 16
                    u = _sk[pl.ds(off, 16)]
                    ix = _si[pl.ds(off, 16)]
                    d = (u >> _shift) & 0xFF
                    base = plsc.load_gather(bs[q], [d])
                    cnt, last = plsc.scan_count(d)
                    dest = base + cnt - 1
                    plsc.store_scatter(_dk, [dest], u)
                    plsc.store_scatter(_di, [dest], ix)
                    plsc.addupdate_scatter(bs[q], [d], cnt, mask=last)
                return cc
            lax.fori_loop(0, SVQ, perm, 0)

        def post(i, cc):
            for j in range(2):
                off = (i * 2 + j) * 16
                u = ckey[pl.ds(off, 16)]
                ix = cidx[pl.ds(off, 16)]
                k = u ^ MIN32
                b = jnp.where(k >= 0, k, M7F8 - k)
                x = plsc.bitcast(b, jnp.float32)
                srow[pl.ds(off, 16)] = 1.0 / (1.0 + jnp.exp(-x))
                lrow[pl.ds(off, 16)] = ix & (NCLS - 1)
                q2 = ix >> 4
                bx0 = plsc.load_gather(boxrow, [2 * q2])
                bx1 = plsc.load_gather(boxrow, [2 * q2 + 1])
                pos = off + iota16
                plsc.store_scatter(brow, [2 * pos], bx0)
                plsc.store_scatter(brow, [2 * pos + 1], bx1)
                crow[pl.ds(off, 16)] = plsc.load_gather(caprow, [q2])
            return cc
        lax.fori_loop(0, SV // 2, post, 0)

        pltpu.sync_copy(srow, scores_hbm.at[row])
        pltpu.sync_copy(lrow, labels_hbm.at[row])
        pltpu.sync_copy(brow, boxes_hbm.at[row])
        pltpu.sync_copy(crow, caps_hbm.at[row])
        return carry

    lax.fori_loop(0, ROWS_PER_W, do_row, 0)


def _topk_sc(bits, boxes_flat, cap_flat):
    mesh = plsc.VectorSubcoreMesh(core_axis_name="c", subcore_axis_name="s")
    fn = pl.kernel(
        _sc_body,
        out_type=[
            jax.ShapeDtypeStruct((B, NQ), jnp.float32),
            jax.ShapeDtypeStruct((B, NQ), jnp.int32),
            jax.ShapeDtypeStruct((B, 2 * NQ), jnp.float32),
            jax.ShapeDtypeStruct((B, NQ), jnp.float32),
        ],
        mesh=mesh,
        scratch_types=[
            pltpu.VMEM((NFLAT,), jnp.int32),
            pltpu.VMEM((NQ + 16,), jnp.int32),
            pltpu.VMEM((NQ + 16,), jnp.int32),
            pltpu.VMEM((NQ,), jnp.int32),
            pltpu.VMEM((NQ,), jnp.int32),
        ] + [pltpu.VMEM((256,), jnp.int32)] * 8 + [
            pltpu.VMEM((4096,), jnp.int32),
            pltpu.VMEM((4096,), jnp.int32),
        ] + [
            pltpu.VMEM((2 * NQ,), jnp.float32),
            pltpu.VMEM((NQ,), jnp.float32),
            pltpu.VMEM((NQ,), jnp.float32),
            pltpu.VMEM((NQ,), jnp.int32),
            pltpu.VMEM((2 * NQ,), jnp.float32),
            pltpu.VMEM((NQ,), jnp.float32),
        ],
        compiler_params=pltpu.CompilerParams(needs_layout_passes=False),
    )
    return fn(bits, boxes_flat, cap_flat)


def kernel(pred_logits, pred_boxes, seq, cap_prob, pred_count, target_sizes):
    boxes_s, cap_full, lens2d = _dense(pred_boxes, seq, cap_prob,
                                       pred_count, target_sizes)
    bits = lax.bitcast_convert_type(pred_logits, jnp.int32).reshape(B, NFLAT)
    scores, labels, boxes_flat, cap_scores = _topk_sc(
        bits, boxes_s, cap_full.reshape(B, NQ))
    boxes = boxes_flat.reshape(B, NQ, 2)
    eseq_lens = lens2d.reshape(B)
    return scores, labels, boxes, cap_scores, eseq_lens
